# baseline (speedup 1.0000x reference)
import functools

import jax
import jax.numpy as jnp
from jax import lax
from jax.experimental import pallas as pl
from jax.experimental.pallas import tpu as pltpu
from jax.experimental.pallas import tpu_sc as plsc

N_TRANS = 320000
N_NEIGH = 10000
D = 128

NC = 2
NS = 16
NW = NC * NS
ROWS_PER_W = N_TRANS // NW
GSZ = 80
GROUPS_PER_CHUNK = 1
CHUNK = GSZ * GROUPS_PER_CHUNK
CHUNKS_PER_W = ROWS_PER_W // CHUNK
NPAIR = (CHUNKS_PER_W - 1) // 2
IDROWS_PER_W = ROWS_PER_W // GSZ
NEIGH_PER_T = N_NEIGH // NS
CNTW = 16

def _mesh():
    return plsc.VectorSubcoreMesh(core_axis_name="c", subcore_axis_name="s",
                                  num_cores=NC, num_subcores=NS)


def _sc_segsum(x, ids2d, zagg, zcnt, ones_cnt):
    def body(x_hbm, ids_hbm, zagg_hbm, zcnt_hbm, ones_hbm,
             agg_out, cnt_out, xa, xb, ia, ib_buf, onesbuf, agg_sh, cnt_sh,
             sem_a, sem_b):
        c = lax.axis_index("c")
        s = lax.axis_index("s")
        pltpu.sync_copy(ones_hbm, onesbuf)
        pltpu.sync_copy(zagg_hbm.at[pl.ds(s * NEIGH_PER_T, NEIGH_PER_T)],
                        agg_sh.at[pl.ds(s * NEIGH_PER_T, NEIGH_PER_T)])
        pltpu.sync_copy(zcnt_hbm.at[pl.ds(s * NEIGH_PER_T, NEIGH_PER_T)],
                        cnt_sh.at[pl.ds(s * NEIGH_PER_T, NEIGH_PER_T)])
        plsc.subcore_barrier()

        w = c * NS + s
        rowbase = w * ROWS_PER_W
        idbase = w * IDROWS_PER_W

        def issue(k, xbuf, idxbuf, sem):
            pltpu.async_copy(x_hbm.at[pl.ds(rowbase + k * CHUNK, CHUNK)],
                             xbuf, sem)
            pltpu.async_copy(ids_hbm.at[pl.ds(idbase + k, 1)], idxbuf, sem)

        def drain(xbuf, idxbuf, sem):
            pltpu.make_async_copy(x_hbm.at[pl.ds(rowbase, CHUNK)],
                                  xbuf, sem).wait()
            pltpu.make_async_copy(ids_hbm.at[pl.ds(idbase, 1)],
                                  idxbuf, sem).wait()

        def scatter(xbuf, idxbuf):
            pltpu.sync_copy(xbuf, agg_sh.at[idxbuf.at[0]], add=True)
            pltpu.sync_copy(onesbuf, cnt_sh.at[idxbuf.at[0]], add=True)

        issue(0, xa, ia, sem_a)

        def pair(i, carry):
            issue(2 * i + 1, xb, ib_buf, sem_b)
            drain(xa, ia, sem_a)
            scatter(xa, ia)
            issue(2 * i + 2, xa, ia, sem_a)
            drain(xb, ib_buf, sem_b)
            scatter(xb, ib_buf)
            return carry

        lax.fori_loop(0, NPAIR, pair, 0)
        drain(xa, ia, sem_a)
        scatter(xa, ia)
        plsc.subcore_barrier()

        pltpu.sync_copy(agg_sh.at[pl.ds(s * NEIGH_PER_T, NEIGH_PER_T)],
                        agg_out.at[c, pl.ds(s * NEIGH_PER_T, NEIGH_PER_T)])
        pltpu.sync_copy(cnt_sh.at[pl.ds(s * NEIGH_PER_T, NEIGH_PER_T)],
                        cnt_out.at[c, pl.ds(s * NEIGH_PER_T, NEIGH_PER_T)])

    f = pl.kernel(
        body,
        out_type=(
            jax.ShapeDtypeStruct((NC, N_NEIGH, D), jnp.float32),
            jax.ShapeDtypeStruct((NC, N_NEIGH, CNTW), jnp.float32),
        ),
        mesh=_mesh(),
        scratch_types=[
            pltpu.VMEM((CHUNK, D), jnp.float32),
            pltpu.VMEM((CHUNK, D), jnp.float32),
            pltpu.VMEM((1, GSZ), jnp.int32),
            pltpu.VMEM((1, GSZ), jnp.int32),
            pltpu.VMEM((GSZ, CNTW), jnp.float32),
            pltpu.VMEM_SHARED((N_NEIGH, D), jnp.float32),
            pltpu.VMEM_SHARED((N_NEIGH, CNTW), jnp.float32),
            pltpu.SemaphoreType.DMA,
            pltpu.SemaphoreType.DMA,
        ],
        compiler_params=pltpu.CompilerParams(use_tc_tiling_on_sc=False),
    )
    return f(x, ids2d, zagg, zcnt, ones_cnt)


P_SPLIT = 2
ROWS_PER_W_G = N_TRANS // P_SPLIT // NW
GSZ_G = 100
CHUNKS_G = ROWS_PER_W_G // GSZ_G
NPAIR_G = (CHUNKS_G - 2) // 2


def _sc_gather(table, ids2d_g, p):
    def body(tab_hbm, ids_hbm, out_hbm, xa, xb, idsbuf, tab_sh, sem_a, sem_b):
        c = lax.axis_index("c")
        s = lax.axis_index("s")
        pltpu.sync_copy(tab_hbm.at[pl.ds(s * NEIGH_PER_T, NEIGH_PER_T)],
                        tab_sh.at[pl.ds(s * NEIGH_PER_T, NEIGH_PER_T)])

        w = c * NS + s
        rowbase = w * ROWS_PER_W_G
        idbase = (p * (N_TRANS // P_SPLIT)) // GSZ_G + w * CHUNKS_G
        pltpu.sync_copy(ids_hbm.at[pl.ds(idbase, CHUNKS_G)], idsbuf)
        plsc.subcore_barrier()

        def gather_write(k, xbuf, sem):
            pltpu.sync_copy(tab_sh.at[idsbuf.at[k]], xbuf)
            pltpu.async_copy(xbuf,
                             out_hbm.at[pl.ds(rowbase + k * GSZ_G, GSZ_G)],
                             sem)

        def drain(xbuf, sem):
            pltpu.make_async_copy(xbuf, out_hbm.at[pl.ds(rowbase, GSZ_G)],
                                  sem).wait()

        gather_write(0, xa, sem_a)

        def pair(i, carry):
            gather_write(2 * i + 1, xb, sem_b)
            drain(xa, sem_a)
            gather_write(2 * i + 2, xa, sem_a)
            drain(xb, sem_b)
            return carry

        lax.fori_loop(0, NPAIR_G, pair, 0)
        gather_write(CHUNKS_G - 1, xb, sem_b)
        drain(xa, sem_a)
        drain(xb, sem_b)

    f = pl.kernel(
        body,
        out_type=jax.ShapeDtypeStruct((N_TRANS // P_SPLIT, D), jnp.bfloat16),
        mesh=_mesh(),
        scratch_types=[
            pltpu.VMEM((GSZ_G, D), jnp.bfloat16),
            pltpu.VMEM((GSZ_G, D), jnp.bfloat16),
            pltpu.VMEM((CHUNKS_G, GSZ_G), jnp.int32),
            pltpu.VMEM_SHARED((N_NEIGH, D), jnp.bfloat16),
            pltpu.SemaphoreType.DMA,
            pltpu.SemaphoreType.DMA,
        ],
        compiler_params=pltpu.CompilerParams(use_tc_tiling_on_sc=False),
    )
    return f(table, ids2d_g)


def _dot_t(a, w):
    return lax.dot_general(a, w, (((1,), (1,)), ((), ())),
                           preferred_element_type=jnp.float32)


def _fuse_body(aggp_ref, cntp_ref, me_ref, wt_ref, bt_ref, wm_ref, bm_ref,
               wf_ref, bf_ref, uf_ref, ufb_ref, fused_ref, fused_bf_ref):
    agg = aggp_ref[0] + aggp_ref[1]
    cnt = cntp_ref[0] + cntp_ref[1]
    n_raw = cnt[:, 0:1]
    n = jnp.maximum(n_raw, 1.0)
    macro_agg = _dot_t(agg, wt_ref[...]) + n_raw * bt_ref[...]
    bottom_up = macro_agg / n
    h_macro = _dot_t(me_ref[...], wm_ref[...]) + bm_ref[...]
    z = (_dot_t(bottom_up, wf_ref[...]) + bf_ref[...]
         + _dot_t(h_macro, uf_ref[...]) + ufb_ref[...])
    fg = jax.nn.sigmoid(z)
    fused = fg * h_macro + (1.0 - fg) * bottom_up
    fused_ref[...] = fused
    fused_bf_ref[...] = fused.astype(jnp.bfloat16)


def _tc_fuse(aggp, cntp, macro_embed, Wt, bt, Wm, bm, Wf, bf, Uf, uf_b):
    B = 2000
    grid = N_NEIGH // B
    w_spec = pl.BlockSpec((D, D), lambda i: (0, 0))
    b_spec = pl.BlockSpec((1, D), lambda i: (0, 0))
    return pl.pallas_call(
        _fuse_body,
        grid=(grid,),
        in_specs=[
            pl.BlockSpec((NC, B, D), lambda i: (0, i, 0)),
            pl.BlockSpec((NC, B, CNTW), lambda i: (0, i, 0)),
            pl.BlockSpec((B, D), lambda i: (i, 0)),
            w_spec, b_spec, w_spec, b_spec, w_spec, b_spec, w_spec, b_spec,
        ],
        out_specs=[pl.BlockSpec((B, D), lambda i: (i, 0)),
                   pl.BlockSpec((B, D), lambda i: (i, 0))],
        out_shape=[jax.ShapeDtypeStruct((N_NEIGH, D), jnp.float32),
                   jax.ShapeDtypeStruct((N_NEIGH, D), jnp.bfloat16)],
        compiler_params=pltpu.CompilerParams(
            dimension_semantics=("arbitrary",)),
    )(aggp, cntp, macro_embed, Wt, bt.reshape(1, D), Wm, bm.reshape(1, D),
      Wf, bf.reshape(1, D), Uf, uf_b.reshape(1, D))


def _final_body(x_ref, al_ref, wt_ref, bt_ref, wg_ref, bg_ref, ug_ref,
                ugb_ref, out_ref):
    h = _dot_t(x_ref[...].astype(jnp.bfloat16), wt_ref[...]) + bt_ref[...]
    al = al_ref[...]
    z = (_dot_t(h.astype(jnp.bfloat16), wg_ref[...]) + bg_ref[...]
         + _dot_t(al, ug_ref[...]) + ugb_ref[...])
    g = jax.nn.sigmoid(z)
    out_ref[...] = g * h + (1.0 - g) * al.astype(jnp.float32)


def _final_body_alias(x_ref, al_ref, prev_ref, wt_ref, bt_ref, wg_ref, bg_ref,
                      ug_ref, ugb_ref, out_ref):
    del prev_ref
    _final_body(x_ref, al_ref, wt_ref, bt_ref, wg_ref, bg_ref, ug_ref,
                ugb_ref, out_ref)


def _tc_final_slice(x, aligned_p, prev, p, Wt, bt, Wg, bg, Ug, ug_b):
    B = 2000
    grid = N_TRANS // P_SPLIT // B
    off = p * grid
    w_spec = pl.BlockSpec((D, D), lambda i: (0, 0))
    b_spec = pl.BlockSpec((1, D), lambda i: (0, 0))
    in_specs = [
        pl.BlockSpec((B, D), lambda i: (i + off, 0)),
        pl.BlockSpec((B, D), lambda i: (i, 0)),
    ]
    args = [x, aligned_p]
    body = _final_body
    kwargs = {}
    if prev is not None:
        in_specs.append(pl.BlockSpec((B, D), lambda i: (0, 0)))
        args.append(prev)
        body = _final_body_alias
        kwargs["input_output_aliases"] = {2: 0}
    return pl.pallas_call(
        body,
        grid=(grid,),
        in_specs=in_specs + [w_spec, b_spec, w_spec, b_spec, w_spec, b_spec],
        out_specs=pl.BlockSpec((B, D), lambda i: (i + off, 0)),
        out_shape=jax.ShapeDtypeStruct((N_TRANS, D), jnp.float32),
        compiler_params=pltpu.CompilerParams(
            dimension_semantics=("arbitrary",)),
        **kwargs,
    )(*args, Wt.astype(jnp.bfloat16), bt.reshape(1, D),
      Wg.astype(jnp.bfloat16), bg.reshape(1, D),
      Ug.astype(jnp.bfloat16), ug_b.reshape(1, D))


def kernel(trans_embed, macro_embed, trans_to_neigh, Wt, bt, Wm, bm, Wg, bg,
           Ug, ug_b, Wf, bf, Uf, uf_b):
    ids32 = trans_to_neigh.astype(jnp.int32)
    ids2d = ids32.reshape(N_TRANS // GSZ, GSZ)
    ids2d_g = ids32.reshape(N_TRANS // GSZ_G, GSZ_G)
    zagg = jnp.zeros((N_NEIGH, D), jnp.float32)
    zcnt = jnp.zeros((N_NEIGH, CNTW), jnp.float32)
    ones_cnt = jnp.zeros((GSZ, CNTW), jnp.float32).at[:, 0].set(1.0)

    aggp, cntp = _sc_segsum(trans_embed, ids2d, zagg, zcnt, ones_cnt)
    fused_macro, fused_bf = _tc_fuse(aggp, cntp, macro_embed, Wt, bt, Wm, bm,
                                     Wf, bf, Uf, uf_b)
    out = None
    for p in range(P_SPLIT):
        aligned_p = _sc_gather(fused_bf, ids2d_g, p)
        out = _tc_final_slice(trans_embed, aligned_p, out, p,
                              Wt, bt, Wg, bg, Ug, ug_b)
    return (out, fused_macro)


# speedup vs baseline: 1.6711x; 1.6711x over previous
import functools

import jax
import jax.numpy as jnp
from jax import lax
from jax.experimental import pallas as pl
from jax.experimental.pallas import tpu as pltpu
from jax.experimental.pallas import tpu_sc as plsc

N_TRANS = 320000
N_NEIGH = 10000
D = 128

NC = 2
NS = 16
NW = NC * NS
ROWS_PER_W = N_TRANS // NW
GSZ = 80
GROUPS_PER_CHUNK = 1
CHUNK = GSZ * GROUPS_PER_CHUNK
CHUNKS_PER_W = ROWS_PER_W // CHUNK
NPAIR = (CHUNKS_PER_W - 1) // 2
IDROWS_PER_W = ROWS_PER_W // GSZ
NEIGH_PER_T = N_NEIGH // NS
CNTW = 16

def _mesh():
    return plsc.VectorSubcoreMesh(core_axis_name="c", subcore_axis_name="s",
                                  num_cores=NC, num_subcores=NS)


def _sc_segsum(x, ids2d, zagg, zcnt, ones_cnt):
    def body(x_hbm, ids_hbm, zagg_hbm, zcnt_hbm, ones_hbm,
             agg_out, cnt_out, xa, xb, ia, ib_buf, onesbuf, agg_sh, cnt_sh,
             sem_a, sem_b):
        c = lax.axis_index("c")
        s = lax.axis_index("s")
        pltpu.sync_copy(ones_hbm, onesbuf)
        pltpu.sync_copy(zagg_hbm.at[pl.ds(s * NEIGH_PER_T, NEIGH_PER_T)],
                        agg_sh.at[pl.ds(s * NEIGH_PER_T, NEIGH_PER_T)])
        pltpu.sync_copy(zcnt_hbm.at[pl.ds(s * NEIGH_PER_T, NEIGH_PER_T)],
                        cnt_sh.at[pl.ds(s * NEIGH_PER_T, NEIGH_PER_T)])
        plsc.subcore_barrier()

        w = c * NS + s
        rowbase = w * ROWS_PER_W
        idbase = w * IDROWS_PER_W

        def issue(k, xbuf, idxbuf, sem):
            pltpu.async_copy(x_hbm.at[pl.ds(rowbase + k * CHUNK, CHUNK)],
                             xbuf, sem)
            pltpu.async_copy(ids_hbm.at[pl.ds(idbase + k, 1)], idxbuf, sem)

        def drain(xbuf, idxbuf, sem):
            pltpu.make_async_copy(x_hbm.at[pl.ds(rowbase, CHUNK)],
                                  xbuf, sem).wait()
            pltpu.make_async_copy(ids_hbm.at[pl.ds(idbase, 1)],
                                  idxbuf, sem).wait()

        def scatter(xbuf, idxbuf):
            pltpu.sync_copy(xbuf, agg_sh.at[idxbuf.at[0]], add=True)
            pltpu.sync_copy(onesbuf, cnt_sh.at[idxbuf.at[0]], add=True)

        issue(0, xa, ia, sem_a)

        def pair(i, carry):
            issue(2 * i + 1, xb, ib_buf, sem_b)
            drain(xa, ia, sem_a)
            scatter(xa, ia)
            issue(2 * i + 2, xa, ia, sem_a)
            drain(xb, ib_buf, sem_b)
            scatter(xb, ib_buf)
            return carry

        lax.fori_loop(0, NPAIR, pair, 0)
        drain(xa, ia, sem_a)
        scatter(xa, ia)
        plsc.subcore_barrier()

        pltpu.sync_copy(agg_sh.at[pl.ds(s * NEIGH_PER_T, NEIGH_PER_T)],
                        agg_out.at[c, pl.ds(s * NEIGH_PER_T, NEIGH_PER_T)])
        pltpu.sync_copy(cnt_sh.at[pl.ds(s * NEIGH_PER_T, NEIGH_PER_T)],
                        cnt_out.at[c, pl.ds(s * NEIGH_PER_T, NEIGH_PER_T)])

    f = pl.kernel(
        body,
        out_type=(
            jax.ShapeDtypeStruct((NC, N_NEIGH, D), jnp.float32),
            jax.ShapeDtypeStruct((NC, N_NEIGH, CNTW), jnp.float32),
        ),
        mesh=_mesh(),
        scratch_types=[
            pltpu.VMEM((CHUNK, D), jnp.float32),
            pltpu.VMEM((CHUNK, D), jnp.float32),
            pltpu.VMEM((1, GSZ), jnp.int32),
            pltpu.VMEM((1, GSZ), jnp.int32),
            pltpu.VMEM((GSZ, CNTW), jnp.float32),
            pltpu.VMEM_SHARED((N_NEIGH, D), jnp.float32),
            pltpu.VMEM_SHARED((N_NEIGH, CNTW), jnp.float32),
            pltpu.SemaphoreType.DMA,
            pltpu.SemaphoreType.DMA,
        ],
        compiler_params=pltpu.CompilerParams(use_tc_tiling_on_sc=False),
    )
    return f(x, ids2d, zagg, zcnt, ones_cnt)


P_SPLIT = 2
ROWS_PER_W_G = N_TRANS // P_SPLIT // NW
GSZ_G = 100
CHUNKS_G = ROWS_PER_W_G // GSZ_G
NPAIR_G = (CHUNKS_G - 2) // 2


def _sc_gather(table, ids2d_g, p):
    def body(tab_hbm, ids_hbm, out_hbm, xa, xb, idsbuf, tab_sh, sem_a, sem_b):
        c = lax.axis_index("c")
        s = lax.axis_index("s")
        pltpu.sync_copy(tab_hbm.at[pl.ds(s * NEIGH_PER_T, NEIGH_PER_T)],
                        tab_sh.at[pl.ds(s * NEIGH_PER_T, NEIGH_PER_T)])

        w = c * NS + s
        rowbase = w * ROWS_PER_W_G
        idbase = (p * (N_TRANS // P_SPLIT)) // GSZ_G + w * CHUNKS_G
        pltpu.sync_copy(ids_hbm.at[pl.ds(idbase, CHUNKS_G)], idsbuf)
        plsc.subcore_barrier()

        def gather_write(k, xbuf, sem):
            pltpu.sync_copy(tab_sh.at[idsbuf.at[k]], xbuf)
            pltpu.async_copy(xbuf,
                             out_hbm.at[pl.ds(rowbase + k * GSZ_G, GSZ_G)],
                             sem)

        def drain(xbuf, sem):
            pltpu.make_async_copy(xbuf, out_hbm.at[pl.ds(rowbase, GSZ_G)],
                                  sem).wait()

        gather_write(0, xa, sem_a)

        def pair(i, carry):
            gather_write(2 * i + 1, xb, sem_b)
            drain(xa, sem_a)
            gather_write(2 * i + 2, xa, sem_a)
            drain(xb, sem_b)
            return carry

        lax.fori_loop(0, NPAIR_G, pair, 0)
        gather_write(CHUNKS_G - 1, xb, sem_b)
        drain(xa, sem_a)
        drain(xb, sem_b)

    f = pl.kernel(
        body,
        out_type=jax.ShapeDtypeStruct((N_TRANS // P_SPLIT, D), jnp.float32),
        mesh=_mesh(),
        scratch_types=[
            pltpu.VMEM((GSZ_G, D), jnp.float32),
            pltpu.VMEM((GSZ_G, D), jnp.float32),
            pltpu.VMEM((CHUNKS_G, GSZ_G), jnp.int32),
            pltpu.VMEM_SHARED((N_NEIGH, D), jnp.float32),
            pltpu.SemaphoreType.DMA,
            pltpu.SemaphoreType.DMA,
        ],
        compiler_params=pltpu.CompilerParams(use_tc_tiling_on_sc=False),
    )
    return f(table, ids2d_g)


def _dot_t(a, w):
    return lax.dot_general(a, w, (((1,), (1,)), ((), ())),
                           preferred_element_type=jnp.float32)


def _fuse_body(aggp_ref, cntp_ref, me_ref, wt_ref, bt_ref, wm_ref, bm_ref,
               wf_ref, bf_ref, uf_ref, ufb_ref, fused_ref):
    agg = aggp_ref[0] + aggp_ref[1]
    cnt = cntp_ref[0] + cntp_ref[1]
    n_raw = cnt[:, 0:1]
    n = jnp.maximum(n_raw, 1.0)
    macro_agg = _dot_t(agg, wt_ref[...]) + n_raw * bt_ref[...]
    bottom_up = macro_agg / n
    h_macro = _dot_t(me_ref[...], wm_ref[...]) + bm_ref[...]
    z = (_dot_t(bottom_up, wf_ref[...]) + bf_ref[...]
         + _dot_t(h_macro, uf_ref[...]) + ufb_ref[...])
    fg = jax.nn.sigmoid(z)
    fused_ref[...] = fg * h_macro + (1.0 - fg) * bottom_up


def _tc_fuse(aggp, cntp, macro_embed, Wt, bt, Wm, bm, Wf, bf, Uf, uf_b):
    B = 2000
    grid = N_NEIGH // B
    w_spec = pl.BlockSpec((D, D), lambda i: (0, 0))
    b_spec = pl.BlockSpec((1, D), lambda i: (0, 0))
    return pl.pallas_call(
        _fuse_body,
        grid=(grid,),
        in_specs=[
            pl.BlockSpec((NC, B, D), lambda i: (0, i, 0)),
            pl.BlockSpec((NC, B, CNTW), lambda i: (0, i, 0)),
            pl.BlockSpec((B, D), lambda i: (i, 0)),
            w_spec, b_spec, w_spec, b_spec, w_spec, b_spec, w_spec, b_spec,
        ],
        out_specs=pl.BlockSpec((B, D), lambda i: (i, 0)),
        out_shape=jax.ShapeDtypeStruct((N_NEIGH, D), jnp.float32),
        compiler_params=pltpu.CompilerParams(
            dimension_semantics=("arbitrary",)),
    )(aggp, cntp, macro_embed, Wt, bt.reshape(1, D), Wm, bm.reshape(1, D),
      Wf, bf.reshape(1, D), Uf, uf_b.reshape(1, D))


def _final_body(x_ref, al_ref, wt_ref, bt_ref, wg_ref, bg_ref, ug_ref,
                ugb_ref, out_ref):
    h = _dot_t(x_ref[...], wt_ref[...]) + bt_ref[...]
    al = al_ref[...]
    z = (_dot_t(h, wg_ref[...]) + bg_ref[...]
         + _dot_t(al, ug_ref[...]) + ugb_ref[...])
    g = jax.nn.sigmoid(z)
    out_ref[...] = g * h + (1.0 - g) * al


def _final_body_alias(x_ref, al_ref, prev_ref, wt_ref, bt_ref, wg_ref, bg_ref,
                      ug_ref, ugb_ref, out_ref):
    del prev_ref
    _final_body(x_ref, al_ref, wt_ref, bt_ref, wg_ref, bg_ref, ug_ref,
                ugb_ref, out_ref)


def _tc_final_slice(x, aligned_p, prev, p, Wt, bt, Wg, bg, Ug, ug_b):
    B = 2000
    grid = N_TRANS // P_SPLIT // B
    off = p * grid
    w_spec = pl.BlockSpec((D, D), lambda i: (0, 0))
    b_spec = pl.BlockSpec((1, D), lambda i: (0, 0))
    in_specs = [
        pl.BlockSpec((B, D), lambda i: (i + off, 0)),
        pl.BlockSpec((B, D), lambda i: (i, 0)),
    ]
    args = [x, aligned_p]
    body = _final_body
    kwargs = {}
    if prev is not None:
        in_specs.append(pl.BlockSpec((B, D), lambda i: (0, 0)))
        args.append(prev)
        body = _final_body_alias
        kwargs["input_output_aliases"] = {2: 0}
    return pl.pallas_call(
        body,
        grid=(grid,),
        in_specs=in_specs + [w_spec, b_spec, w_spec, b_spec, w_spec, b_spec],
        out_specs=pl.BlockSpec((B, D), lambda i: (i + off, 0)),
        out_shape=jax.ShapeDtypeStruct((N_TRANS, D), jnp.float32),
        compiler_params=pltpu.CompilerParams(
            dimension_semantics=("arbitrary",)),
        **kwargs,
    )(*args, Wt, bt.reshape(1, D), Wg, bg.reshape(1, D), Ug,
      ug_b.reshape(1, D))


def kernel(trans_embed, macro_embed, trans_to_neigh, Wt, bt, Wm, bm, Wg, bg,
           Ug, ug_b, Wf, bf, Uf, uf_b):
    ids32 = trans_to_neigh.astype(jnp.int32)
    ids2d = ids32.reshape(N_TRANS // GSZ, GSZ)
    ids2d_g = ids32.reshape(N_TRANS // GSZ_G, GSZ_G)
    zagg = jnp.zeros((N_NEIGH, D), jnp.float32)
    zcnt = jnp.zeros((N_NEIGH, CNTW), jnp.float32)
    ones_cnt = jnp.zeros((GSZ, CNTW), jnp.float32).at[:, 0].set(1.0)

    aggp, cntp = _sc_segsum(trans_embed, ids2d, zagg, zcnt, ones_cnt)
    fused_macro = _tc_fuse(aggp, cntp, macro_embed, Wt, bt, Wm, bm,
                           Wf, bf, Uf, uf_b)
    out = None
    for p in range(P_SPLIT):
        aligned_p = _sc_gather(fused_macro, ids2d_g, p)
        out = _tc_final_slice(trans_embed, aligned_p, out, p,
                              Wt, bt, Wg, bg, Ug, ug_b)
    return (out, fused_macro)


# speedup vs baseline: 1.8745x; 1.1217x over previous
import functools

import jax
import jax.numpy as jnp
from jax import lax
from jax.experimental import pallas as pl
from jax.experimental.pallas import tpu as pltpu
from jax.experimental.pallas import tpu_sc as plsc

N_TRANS = 320000
N_NEIGH = 10000
D = 128

NC = 2
NS = 16
NW = NC * NS
ROWS_PER_W = N_TRANS // NW
GSZ = 80
GROUPS_PER_CHUNK = 1
CHUNK = GSZ * GROUPS_PER_CHUNK
CHUNKS_PER_W = ROWS_PER_W // CHUNK
NPAIR = (CHUNKS_PER_W - 1) // 2
IDROWS_PER_W = ROWS_PER_W // GSZ
NEIGH_PER_T = N_NEIGH // NS
CNTW = 16

def _mesh():
    return plsc.VectorSubcoreMesh(core_axis_name="c", subcore_axis_name="s",
                                  num_cores=NC, num_subcores=NS)


def _sc_segsum(x, ids2d, zagg, zcnt, ones_cnt):
    def body(x_hbm, ids_hbm, zagg_hbm, zcnt_hbm, ones_hbm,
             agg_out, cnt_out, xa, xb, ia, ib_buf, onesbuf, agg_sh, cnt_sh,
             sem_a, sem_b):
        c = lax.axis_index("c")
        s = lax.axis_index("s")
        pltpu.sync_copy(ones_hbm, onesbuf)
        pltpu.sync_copy(zagg_hbm.at[pl.ds(s * NEIGH_PER_T, NEIGH_PER_T)],
                        agg_sh.at[pl.ds(s * NEIGH_PER_T, NEIGH_PER_T)])
        pltpu.sync_copy(zcnt_hbm.at[pl.ds(s * NEIGH_PER_T, NEIGH_PER_T)],
                        cnt_sh.at[pl.ds(s * NEIGH_PER_T, NEIGH_PER_T)])
        plsc.subcore_barrier()

        w = c * NS + s
        rowbase = w * ROWS_PER_W
        idbase = w * IDROWS_PER_W

        def issue(k, xbuf, idxbuf, sem):
            pltpu.async_copy(x_hbm.at[pl.ds(rowbase + k * CHUNK, CHUNK)],
                             xbuf, sem)
            pltpu.async_copy(ids_hbm.at[pl.ds(idbase + k, 1)], idxbuf, sem)

        def drain(xbuf, idxbuf, sem):
            pltpu.make_async_copy(x_hbm.at[pl.ds(rowbase, CHUNK)],
                                  xbuf, sem).wait()
            pltpu.make_async_copy(ids_hbm.at[pl.ds(idbase, 1)],
                                  idxbuf, sem).wait()

        def scatter(xbuf, idxbuf):
            pltpu.sync_copy(xbuf, agg_sh.at[idxbuf.at[0]], add=True)
            pltpu.sync_copy(onesbuf, cnt_sh.at[idxbuf.at[0]], add=True)

        issue(0, xa, ia, sem_a)

        def pair(i, carry):
            issue(2 * i + 1, xb, ib_buf, sem_b)
            drain(xa, ia, sem_a)
            scatter(xa, ia)
            issue(2 * i + 2, xa, ia, sem_a)
            drain(xb, ib_buf, sem_b)
            scatter(xb, ib_buf)
            return carry

        lax.fori_loop(0, NPAIR, pair, 0)
        drain(xa, ia, sem_a)
        scatter(xa, ia)
        plsc.subcore_barrier()

        pltpu.sync_copy(agg_sh.at[pl.ds(s * NEIGH_PER_T, NEIGH_PER_T)],
                        agg_out.at[c, pl.ds(s * NEIGH_PER_T, NEIGH_PER_T)])
        pltpu.sync_copy(cnt_sh.at[pl.ds(s * NEIGH_PER_T, NEIGH_PER_T)],
                        cnt_out.at[c, pl.ds(s * NEIGH_PER_T, NEIGH_PER_T)])

    f = pl.kernel(
        body,
        out_type=(
            jax.ShapeDtypeStruct((NC, N_NEIGH, D), jnp.float32),
            jax.ShapeDtypeStruct((NC, N_NEIGH, CNTW), jnp.float32),
        ),
        mesh=_mesh(),
        scratch_types=[
            pltpu.VMEM((CHUNK, D), jnp.float32),
            pltpu.VMEM((CHUNK, D), jnp.float32),
            pltpu.VMEM((1, GSZ), jnp.int32),
            pltpu.VMEM((1, GSZ), jnp.int32),
            pltpu.VMEM((GSZ, CNTW), jnp.float32),
            pltpu.VMEM_SHARED((N_NEIGH, D), jnp.float32),
            pltpu.VMEM_SHARED((N_NEIGH, CNTW), jnp.float32),
            pltpu.SemaphoreType.DMA,
            pltpu.SemaphoreType.DMA,
        ],
        compiler_params=pltpu.CompilerParams(use_tc_tiling_on_sc=False),
    )
    return f(x, ids2d, zagg, zcnt, ones_cnt)


P_SPLIT = 2
ROWS_PER_W_G = N_TRANS // P_SPLIT // NW
GSZ_G = 100
CHUNKS_G = ROWS_PER_W_G // GSZ_G
NPAIR_G = (CHUNKS_G - 2) // 2


def _sc_gather(table, ids2d_g, p):
    def body(tab_hbm, ids_hbm, out_hbm, xa, xb, idsbuf, tab_sh, sem_a, sem_b):
        c = lax.axis_index("c")
        s = lax.axis_index("s")
        pltpu.sync_copy(tab_hbm.at[pl.ds(s * NEIGH_PER_T, NEIGH_PER_T)],
                        tab_sh.at[pl.ds(s * NEIGH_PER_T, NEIGH_PER_T)])

        w = c * NS + s
        rowbase = w * ROWS_PER_W_G
        idbase = (p * (N_TRANS // P_SPLIT)) // GSZ_G + w * CHUNKS_G
        pltpu.sync_copy(ids_hbm.at[pl.ds(idbase, CHUNKS_G)], idsbuf)
        plsc.subcore_barrier()

        def gather_write(k, xbuf, sem):
            pltpu.sync_copy(tab_sh.at[idsbuf.at[k]], xbuf)
            pltpu.async_copy(xbuf,
                             out_hbm.at[pl.ds(rowbase + k * GSZ_G, GSZ_G)],
                             sem)

        def drain(xbuf, sem):
            pltpu.make_async_copy(xbuf, out_hbm.at[pl.ds(rowbase, GSZ_G)],
                                  sem).wait()

        gather_write(0, xa, sem_a)

        def pair(i, carry):
            gather_write(2 * i + 1, xb, sem_b)
            drain(xa, sem_a)
            gather_write(2 * i + 2, xa, sem_a)
            drain(xb, sem_b)
            return carry

        lax.fori_loop(0, NPAIR_G, pair, 0)
        gather_write(CHUNKS_G - 1, xb, sem_b)
        drain(xa, sem_a)
        drain(xb, sem_b)

    f = pl.kernel(
        body,
        out_type=jax.ShapeDtypeStruct((N_TRANS // P_SPLIT, D), jnp.float32),
        mesh=_mesh(),
        scratch_types=[
            pltpu.VMEM((GSZ_G, D), jnp.float32),
            pltpu.VMEM((GSZ_G, D), jnp.float32),
            pltpu.VMEM((CHUNKS_G, GSZ_G), jnp.int32),
            pltpu.VMEM_SHARED((N_NEIGH, D), jnp.float32),
            pltpu.SemaphoreType.DMA,
            pltpu.SemaphoreType.DMA,
        ],
        compiler_params=pltpu.CompilerParams(use_tc_tiling_on_sc=False),
    )
    return f(table, ids2d_g)


def _dot_t(a, w):
    return lax.dot_general(a, w, (((1,), (1,)), ((), ())),
                           preferred_element_type=jnp.float32)


def _fuse_body(aggp_ref, cntp_ref, me_ref, wt_ref, bt_ref, wm_ref, bm_ref,
               wf_ref, bf_ref, uf_ref, ufb_ref, fused_ref):
    agg = aggp_ref[0] + aggp_ref[1]
    cnt = cntp_ref[0] + cntp_ref[1]
    n_raw = cnt[:, 0:1]
    n = jnp.maximum(n_raw, 1.0)
    macro_agg = _dot_t(agg, wt_ref[...]) + n_raw * bt_ref[...]
    bottom_up = macro_agg / n
    h_macro = _dot_t(me_ref[...], wm_ref[...]) + bm_ref[...]
    z = (_dot_t(bottom_up, wf_ref[...]) + bf_ref[...]
         + _dot_t(h_macro, uf_ref[...]) + ufb_ref[...])
    fg = jax.nn.sigmoid(z)
    fused_ref[...] = fg * h_macro + (1.0 - fg) * bottom_up


def _tc_fuse(aggp, cntp, macro_embed, Wt, bt, Wm, bm, Wf, bf, Uf, uf_b):
    B = 2000
    grid = N_NEIGH // B
    w_spec = pl.BlockSpec((D, D), lambda i: (0, 0))
    b_spec = pl.BlockSpec((1, D), lambda i: (0, 0))
    return pl.pallas_call(
        _fuse_body,
        grid=(grid,),
        in_specs=[
            pl.BlockSpec((NC, B, D), lambda i: (0, i, 0)),
            pl.BlockSpec((NC, B, CNTW), lambda i: (0, i, 0)),
            pl.BlockSpec((B, D), lambda i: (i, 0)),
            w_spec, b_spec, w_spec, b_spec, w_spec, b_spec, w_spec, b_spec,
        ],
        out_specs=pl.BlockSpec((B, D), lambda i: (i, 0)),
        out_shape=jax.ShapeDtypeStruct((N_NEIGH, D), jnp.float32),
        compiler_params=pltpu.CompilerParams(
            dimension_semantics=("arbitrary",)),
    )(aggp, cntp, macro_embed, Wt, bt.reshape(1, D), Wm, bm.reshape(1, D),
      Wf, bf.reshape(1, D), Uf, uf_b.reshape(1, D))


def _final_body(x_ref, al_ref, wt_ref, bt_ref, wg_ref, bg_ref, ug_ref,
                ugb_ref, out_ref):
    h = _dot_t(x_ref[...], wt_ref[...]) + bt_ref[...]
    al = al_ref[...]
    z = (_dot_t(h, wg_ref[...]) + bg_ref[...]
         + _dot_t(al, ug_ref[...]) + ugb_ref[...])
    g = jax.nn.sigmoid(z)
    out_ref[...] = g * h + (1.0 - g) * al


def _final_body_alias(x_ref, al_ref, prev_ref, wt_ref, bt_ref, wg_ref, bg_ref,
                      ug_ref, ugb_ref, out_ref):
    del prev_ref
    _final_body(x_ref, al_ref, wt_ref, bt_ref, wg_ref, bg_ref, ug_ref,
                ugb_ref, out_ref)


def _tc_final_slice(x, aligned_p, prev, p, Wt, bt, Wg, bg, Ug, ug_b):
    B = 5000
    grid = N_TRANS // P_SPLIT // B
    off = p * grid
    w_spec = pl.BlockSpec((D, D), lambda i: (0, 0))
    b_spec = pl.BlockSpec((1, D), lambda i: (0, 0))
    in_specs = [
        pl.BlockSpec((B, D), lambda i: (i + off, 0)),
        pl.BlockSpec((B, D), lambda i: (i, 0)),
    ]
    args = [x, aligned_p]
    body = _final_body
    kwargs = {}
    if prev is not None:
        in_specs.append(pl.BlockSpec((B, D), lambda i: (0, 0)))
        args.append(prev)
        body = _final_body_alias
        kwargs["input_output_aliases"] = {2: 0}
    return pl.pallas_call(
        body,
        grid=(grid,),
        in_specs=in_specs + [w_spec, b_spec, w_spec, b_spec, w_spec, b_spec],
        out_specs=pl.BlockSpec((B, D), lambda i: (i + off, 0)),
        out_shape=jax.ShapeDtypeStruct((N_TRANS, D), jnp.float32),
        compiler_params=pltpu.CompilerParams(
            dimension_semantics=("arbitrary",)),
        **kwargs,
    )(*args, Wt, bt.reshape(1, D), Wg, bg.reshape(1, D), Ug,
      ug_b.reshape(1, D))


def kernel(trans_embed, macro_embed, trans_to_neigh, Wt, bt, Wm, bm, Wg, bg,
           Ug, ug_b, Wf, bf, Uf, uf_b):
    ids32 = trans_to_neigh.astype(jnp.int32)
    ids2d = ids32.reshape(N_TRANS // GSZ, GSZ)
    ids2d_g = ids32.reshape(N_TRANS // GSZ_G, GSZ_G)
    zagg = jnp.zeros((N_NEIGH, D), jnp.float32)
    zcnt = jnp.zeros((N_NEIGH, CNTW), jnp.float32)
    ones_cnt = jnp.zeros((GSZ, CNTW), jnp.float32).at[:, 0].set(1.0)

    aggp, cntp = _sc_segsum(trans_embed, ids2d, zagg, zcnt, ones_cnt)
    fused_macro = _tc_fuse(aggp, cntp, macro_embed, Wt, bt, Wm, bm,
                           Wf, bf, Uf, uf_b)
    out = None
    for p in range(P_SPLIT):
        aligned_p = _sc_gather(fused_macro, ids2d_g, p)
        out = _tc_final_slice(trans_embed, aligned_p, out, p,
                              Wt, bt, Wg, bg, Ug, ug_b)
    return (out, fused_macro)


# speedup vs baseline: 1.9512x; 1.0409x over previous
import functools

import jax
import jax.numpy as jnp
from jax import lax
from jax.experimental import pallas as pl
from jax.experimental.pallas import tpu as pltpu
from jax.experimental.pallas import tpu_sc as plsc

N_TRANS = 320000
N_NEIGH = 10000
D = 128

NC = 2
NS = 16
NW = NC * NS
ROWS_PER_W = N_TRANS // NW
GSZ = 80
GROUPS_PER_CHUNK = 1
CHUNK = GSZ * GROUPS_PER_CHUNK
CHUNKS_PER_W = ROWS_PER_W // CHUNK
NPAIR = (CHUNKS_PER_W - 1) // 2
IDROWS_PER_W = ROWS_PER_W // GSZ
NEIGH_PER_T = N_NEIGH // NS
CNTW = 16

def _mesh():
    return plsc.VectorSubcoreMesh(core_axis_name="c", subcore_axis_name="s",
                                  num_cores=NC, num_subcores=NS)


def _sc_segsum(x, ids2d, zagg, zcnt, ones_cnt):
    def body(x_hbm, ids_hbm, zagg_hbm, zcnt_hbm, ones_hbm,
             agg_out, cnt_out, xa, xb, ia, ib_buf, onesbuf, agg_sh, cnt_sh,
             sem_a, sem_b):
        c = lax.axis_index("c")
        s = lax.axis_index("s")
        pltpu.sync_copy(ones_hbm, onesbuf)
        pltpu.sync_copy(zagg_hbm.at[pl.ds(s * NEIGH_PER_T, NEIGH_PER_T)],
                        agg_sh.at[pl.ds(s * NEIGH_PER_T, NEIGH_PER_T)])
        pltpu.sync_copy(zcnt_hbm.at[pl.ds(s * NEIGH_PER_T, NEIGH_PER_T)],
                        cnt_sh.at[pl.ds(s * NEIGH_PER_T, NEIGH_PER_T)])
        plsc.subcore_barrier()

        w = c * NS + s
        rowbase = w * ROWS_PER_W
        idbase = w * IDROWS_PER_W

        def issue(k, xbuf, idxbuf, sem):
            pltpu.async_copy(x_hbm.at[pl.ds(rowbase + k * CHUNK, CHUNK)],
                             xbuf, sem)
            pltpu.async_copy(ids_hbm.at[pl.ds(idbase + k, 1)], idxbuf, sem)

        def drain(xbuf, idxbuf, sem):
            pltpu.make_async_copy(x_hbm.at[pl.ds(rowbase, CHUNK)],
                                  xbuf, sem).wait()
            pltpu.make_async_copy(ids_hbm.at[pl.ds(idbase, 1)],
                                  idxbuf, sem).wait()

        def scatter(xbuf, idxbuf):
            pltpu.sync_copy(xbuf, agg_sh.at[idxbuf.at[0]], add=True)
            pltpu.sync_copy(onesbuf, cnt_sh.at[idxbuf.at[0]], add=True)

        issue(0, xa, ia, sem_a)

        def pair(i, carry):
            issue(2 * i + 1, xb, ib_buf, sem_b)
            drain(xa, ia, sem_a)
            scatter(xa, ia)
            issue(2 * i + 2, xa, ia, sem_a)
            drain(xb, ib_buf, sem_b)
            scatter(xb, ib_buf)
            return carry

        lax.fori_loop(0, NPAIR, pair, 0)
        drain(xa, ia, sem_a)
        scatter(xa, ia)
        plsc.subcore_barrier()

        pltpu.sync_copy(agg_sh.at[pl.ds(s * NEIGH_PER_T, NEIGH_PER_T)],
                        agg_out.at[c, pl.ds(s * NEIGH_PER_T, NEIGH_PER_T)])
        pltpu.sync_copy(cnt_sh.at[pl.ds(s * NEIGH_PER_T, NEIGH_PER_T)],
                        cnt_out.at[c, pl.ds(s * NEIGH_PER_T, NEIGH_PER_T)])

    f = pl.kernel(
        body,
        out_type=(
            jax.ShapeDtypeStruct((NC, N_NEIGH, D), jnp.float32),
            jax.ShapeDtypeStruct((NC, N_NEIGH, CNTW), jnp.float32),
        ),
        mesh=_mesh(),
        scratch_types=[
            pltpu.VMEM((CHUNK, D), jnp.float32),
            pltpu.VMEM((CHUNK, D), jnp.float32),
            pltpu.VMEM((1, GSZ), jnp.int32),
            pltpu.VMEM((1, GSZ), jnp.int32),
            pltpu.VMEM((GSZ, CNTW), jnp.float32),
            pltpu.VMEM_SHARED((N_NEIGH, D), jnp.float32),
            pltpu.VMEM_SHARED((N_NEIGH, CNTW), jnp.float32),
            pltpu.SemaphoreType.DMA,
            pltpu.SemaphoreType.DMA,
        ],
        compiler_params=pltpu.CompilerParams(use_tc_tiling_on_sc=False),
    )
    return f(x, ids2d, zagg, zcnt, ones_cnt)


P_SPLIT = 2
ROWS_PER_W_G = N_TRANS // P_SPLIT // NW
GSZ_G = 100
CHUNKS_G = ROWS_PER_W_G // GSZ_G
NPAIR_G = (CHUNKS_G - 2) // 2


def _sc_gather(table, ids2d_g, p):
    def body(tab_hbm, ids_hbm, out_hbm, xa, xb, idsbuf, tab_sh, sem_a, sem_b):
        c = lax.axis_index("c")
        s = lax.axis_index("s")
        pltpu.sync_copy(tab_hbm.at[pl.ds(s * NEIGH_PER_T, NEIGH_PER_T)],
                        tab_sh.at[pl.ds(s * NEIGH_PER_T, NEIGH_PER_T)])

        w = c * NS + s
        rowbase = w * ROWS_PER_W_G
        idbase = (p * (N_TRANS // P_SPLIT)) // GSZ_G + w * CHUNKS_G
        pltpu.sync_copy(ids_hbm.at[pl.ds(idbase, CHUNKS_G)], idsbuf)
        plsc.subcore_barrier()

        def gather_write(k, xbuf, sem):
            pltpu.sync_copy(tab_sh.at[idsbuf.at[k]], xbuf)
            pltpu.async_copy(xbuf,
                             out_hbm.at[pl.ds(rowbase + k * GSZ_G, GSZ_G)],
                             sem)

        def drain(xbuf, sem):
            pltpu.make_async_copy(xbuf, out_hbm.at[pl.ds(rowbase, GSZ_G)],
                                  sem).wait()

        gather_write(0, xa, sem_a)

        def pair(i, carry):
            gather_write(2 * i + 1, xb, sem_b)
            drain(xa, sem_a)
            gather_write(2 * i + 2, xa, sem_a)
            drain(xb, sem_b)
            return carry

        lax.fori_loop(0, NPAIR_G, pair, 0)
        gather_write(CHUNKS_G - 1, xb, sem_b)
        drain(xa, sem_a)
        drain(xb, sem_b)

    f = pl.kernel(
        body,
        out_type=jax.ShapeDtypeStruct((N_TRANS // P_SPLIT, D), jnp.float32),
        mesh=_mesh(),
        scratch_types=[
            pltpu.VMEM((GSZ_G, D), jnp.float32),
            pltpu.VMEM((GSZ_G, D), jnp.float32),
            pltpu.VMEM((CHUNKS_G, GSZ_G), jnp.int32),
            pltpu.VMEM_SHARED((N_NEIGH, D), jnp.float32),
            pltpu.SemaphoreType.DMA,
            pltpu.SemaphoreType.DMA,
        ],
        compiler_params=pltpu.CompilerParams(use_tc_tiling_on_sc=False),
    )
    return f(table, ids2d_g)


def _dot_t(a, w):
    return lax.dot_general(a, w, (((1,), (1,)), ((), ())),
                           preferred_element_type=jnp.float32)


def _fuse_body(aggp_ref, cntp_ref, me_ref, wt_ref, bt_ref, wm_ref, bm_ref,
               wf_ref, bf_ref, uf_ref, ufb_ref, fused_ref):
    agg = aggp_ref[0] + aggp_ref[1]
    cnt = cntp_ref[0] + cntp_ref[1]
    n_raw = cnt[:, 0:1]
    n = jnp.maximum(n_raw, 1.0)
    macro_agg = _dot_t(agg, wt_ref[...]) + n_raw * bt_ref[...]
    bottom_up = macro_agg / n
    h_macro = _dot_t(me_ref[...], wm_ref[...]) + bm_ref[...]
    z = (_dot_t(bottom_up, wf_ref[...]) + bf_ref[...]
         + _dot_t(h_macro, uf_ref[...]) + ufb_ref[...])
    fg = jax.nn.sigmoid(z)
    fused_ref[...] = fg * h_macro + (1.0 - fg) * bottom_up


def _tc_fuse(aggp, cntp, macro_embed, Wt, bt, Wm, bm, Wf, bf, Uf, uf_b):
    B = 2000
    grid = N_NEIGH // B
    w_spec = pl.BlockSpec((D, D), lambda i: (0, 0))
    b_spec = pl.BlockSpec((1, D), lambda i: (0, 0))
    return pl.pallas_call(
        _fuse_body,
        grid=(grid,),
        in_specs=[
            pl.BlockSpec((NC, B, D), lambda i: (0, i, 0)),
            pl.BlockSpec((NC, B, CNTW), lambda i: (0, i, 0)),
            pl.BlockSpec((B, D), lambda i: (i, 0)),
            w_spec, b_spec, w_spec, b_spec, w_spec, b_spec, w_spec, b_spec,
        ],
        out_specs=pl.BlockSpec((B, D), lambda i: (i, 0)),
        out_shape=jax.ShapeDtypeStruct((N_NEIGH, D), jnp.float32),
        compiler_params=pltpu.CompilerParams(
            dimension_semantics=("arbitrary",)),
    )(aggp, cntp, macro_embed, Wt, bt.reshape(1, D), Wm, bm.reshape(1, D),
      Wf, bf.reshape(1, D), Uf, uf_b.reshape(1, D))


def _final_body(x_ref, al_ref, wt_ref, bt_ref, wg_ref, bg_ref, ug_ref,
                ugb_ref, out_ref):
    h = _dot_t(x_ref[...], wt_ref[...]) + bt_ref[...]
    al = al_ref[...]
    z = (_dot_t(h, wg_ref[...]) + bg_ref[...]
         + _dot_t(al, ug_ref[...]) + ugb_ref[...])
    g = jax.nn.sigmoid(z)
    out_ref[...] = g * h + (1.0 - g) * al


def _final_body_alias(x_ref, al_ref, prev_ref, wt_ref, bt_ref, wg_ref, bg_ref,
                      ug_ref, ugb_ref, out_ref):
    del prev_ref
    _final_body(x_ref, al_ref, wt_ref, bt_ref, wg_ref, bg_ref, ug_ref,
                ugb_ref, out_ref)


def _tc_final_slice(x, aligned_p, prev, p, Wt, bt, Wg, bg, Ug, ug_b):
    B = 10000
    grid = N_TRANS // P_SPLIT // B
    off = p * grid
    w_spec = pl.BlockSpec((D, D), lambda i: (0, 0))
    b_spec = pl.BlockSpec((1, D), lambda i: (0, 0))
    in_specs = [
        pl.BlockSpec((B, D), lambda i: (i + off, 0)),
        pl.BlockSpec((B, D), lambda i: (i, 0)),
    ]
    args = [x, aligned_p]
    body = _final_body
    kwargs = {}
    if prev is not None:
        in_specs.append(pl.BlockSpec((B, D), lambda i: (0, 0)))
        args.append(prev)
        body = _final_body_alias
        kwargs["input_output_aliases"] = {2: 0}
    return pl.pallas_call(
        body,
        grid=(grid,),
        in_specs=in_specs + [w_spec, b_spec, w_spec, b_spec, w_spec, b_spec],
        out_specs=pl.BlockSpec((B, D), lambda i: (i + off, 0)),
        out_shape=jax.ShapeDtypeStruct((N_TRANS, D), jnp.float32),
        compiler_params=pltpu.CompilerParams(
            dimension_semantics=("arbitrary",)),
        **kwargs,
    )(*args, Wt, bt.reshape(1, D), Wg, bg.reshape(1, D), Ug,
      ug_b.reshape(1, D))


def kernel(trans_embed, macro_embed, trans_to_neigh, Wt, bt, Wm, bm, Wg, bg,
           Ug, ug_b, Wf, bf, Uf, uf_b):
    ids32 = trans_to_neigh.astype(jnp.int32)
    ids2d = ids32.reshape(N_TRANS // GSZ, GSZ)
    ids2d_g = ids32.reshape(N_TRANS // GSZ_G, GSZ_G)
    zagg = jnp.zeros((N_NEIGH, D), jnp.float32)
    zcnt = jnp.zeros((N_NEIGH, CNTW), jnp.float32)
    ones_cnt = jnp.zeros((GSZ, CNTW), jnp.float32).at[:, 0].set(1.0)

    aggp, cntp = _sc_segsum(trans_embed, ids2d, zagg, zcnt, ones_cnt)
    fused_macro = _tc_fuse(aggp, cntp, macro_embed, Wt, bt, Wm, bm,
                           Wf, bf, Uf, uf_b)
    out = None
    for p in range(P_SPLIT):
        aligned_p = _sc_gather(fused_macro, ids2d_g, p)
        out = _tc_final_slice(trans_embed, aligned_p, out, p,
                              Wt, bt, Wg, bg, Ug, ug_b)
    return (out, fused_macro)
